# baseline (speedup 1.0000x reference)
import functools

import jax
import jax.numpy as jnp
from jax import lax
from jax.experimental import pallas as pl
from jax.experimental.pallas import tpu as pltpu
from jax.experimental.pallas import tpu_sc as plsc

B = 1024
V = 100000
NC, NS, L = 2, 16, 16
NW = NC * NS
ROWS = B // NW
RG = 8
NG = ROWS // RG
VA = (V // 128) * 128
VT = V - VA
CW = 11 * 128
NCH = VA // CW
NU = NG * NCH
KS = CW // 32
NB = 4


@functools.cache
def _build_head():
    mesh = plsc.VectorSubcoreMesh(core_axis_name="c", subcore_axis_name="s",
                                  num_cores=NC, num_subcores=NS)
    return functools.partial(
        pl.kernel,
        out_type=(jax.ShapeDtypeStruct((B, V), jnp.float32),
                  jax.ShapeDtypeStruct((B, 2), jnp.float32)),
        mesh=mesh,
        compiler_params=pltpu.CompilerParams(needs_layout_passes=False),
        scratch_types=[
            [pltpu.VMEM((RG, CW), jnp.float32) for _ in range(NB)],
            pltpu.VMEM((ROWS, VT), jnp.float32),
            pltpu.VMEM((ROWS,), jnp.int32),
            pltpu.VMEM((ROWS, 2), jnp.float32),
            pltpu.VMEM((ROWS, 128), jnp.float32),
            [pltpu.SemaphoreType.DMA for _ in range(NB)],
            [pltpu.SemaphoreType.DMA for _ in range(NB)],
        ],
    )(_head_body)


def _head_body(x_hbm, xt_hbm, gps_hbm, xout_hbm, out_hbm, bufs, xtbuf,
               idxbuf, gpsbuf, rowbuf, isems, wsems):
    wid = lax.axis_index("s") * NC + lax.axis_index("c")
    row0 = wid * ROWS
    lane = lax.iota(jnp.int32, L)

    def dma_in(u, b):
        g, c = u // NCH, u % NCH
        return pltpu.make_async_copy(
            x_hbm.at[pl.ds(row0 + g * RG, RG), pl.ds(c * CW, CW)],
            bufs[b], isems[b])

    def dma_out(u, b):
        g, c = u // NCH, u % NCH
        return pltpu.make_async_copy(
            bufs[b],
            xout_hbm.at[pl.ds(row0 + g * RG, RG), pl.ds(c * CW, CW)],
            wsems[b])

    neg = jnp.full((L,), -jnp.inf, jnp.float32)
    zero = jnp.zeros((L,), jnp.int32)
    init_acc = ((neg, neg), (zero, zero))

    def scan_chunk(buf, c, acc):
        def body(k, a):
            iv = jnp.full((L,), c * KS + k, jnp.int32)
            out = []
            for r in range(RG):
                (v0, v1), (t0, t1) = a[r]
                x0 = buf[r, pl.ds(k * 32, L)]
                x1 = buf[r, pl.ds(k * 32 + L, L)]
                m0 = x0 > v0
                m1 = x1 > v1
                out.append(((jnp.where(m0, x0, v0), jnp.where(m1, x1, v1)),
                            (jnp.where(m0, iv, t0), jnp.where(m1, iv, t1))))
            return tuple(out)

        return lax.fori_loop(0, KS, body, acc)

    def epilogue(g, acc):
        for r in range(RG):
            row_l = g * RG + r
            (v0, v1), (t0, t1) = acc[r]
            pairs = [
                (v0, t0 * 32 + lane),
                (v1, t1 * 32 + (lane + L)),
                (xtbuf[row_l, pl.ds(0, L)], lane + VA),
                (xtbuf[row_l, pl.ds(L, L)], lane + (VA + L)),
            ]
            bv, bi = pairs[0]
            for v, i in pairs[1:]:
                take = (v > bv) | ((v == bv) & (i < bi))
                bv = jnp.where(take, v, bv)
                bi = jnp.where(take, i, bi)
            m = jnp.float32(-jnp.inf)
            mi = jnp.int32(V)
            for l in range(L):
                v, i = bv[l], bi[l]
                take = (v > m) | ((v == m) & (i < mi))
                m = jnp.where(take, v, m)
                mi = jnp.where(take, i, mi)
            plsc.store_scatter(idxbuf, [jnp.full((L,), row_l, jnp.int32)],
                               jnp.full((L,), mi, jnp.int32), mask=lane == 0)

    pltpu.sync_copy(xt_hbm.at[pl.ds(row0, ROWS)], xtbuf)
    dma_in(0, 0).start()
    dma_in(1, 1).start()

    def unit(u, j, acc):
        g, c = u // NCH, u % NCH
        dma_in(u, j).wait()
        acc = scan_chunk(bufs[j], c, acc)
        dma_out(u, j).start()

        jn = (j + 2) % NB

        @pl.when((u >= 2) & (u + 2 < NU))
        def _drain_write():
            dma_out(u - 2, jn).wait()

        @pl.when(u + 2 < NU)
        def _prefetch():
            dma_in(u + 2, jn).start()

        @pl.when(c == NCH - 1)
        def _finish():
            epilogue(g, acc)

        reset = jnp.full((L,), c == NCH - 1)
        return tuple(
            ((jnp.where(reset, neg, v0), jnp.where(reset, neg, v1)),
             (jnp.where(reset, zero, t0), jnp.where(reset, zero, t1)))
            for (v0, v1), (t0, t1) in acc)

    def quad_body(i, acc):
        for j in range(NB):
            acc = unit(NB * i + j, j, acc)
        return acc

    lax.fori_loop(0, NU // NB, quad_body, (init_acc,) * RG)

    for u in range(NU - NB, NU):
        dma_out(u, u % NB).wait()
    pltpu.sync_copy(xtbuf, xout_hbm.at[pl.ds(row0, ROWS), pl.ds(VA, VT)])

    pltpu.sync_copy(gps_hbm.at[idxbuf], rowbuf)
    ones_b = jnp.full((L,), True)
    zero_i = jnp.zeros((L,), jnp.int32)
    one_i = jnp.full((L,), 1, jnp.int32)
    for h in range(2):
        rvec = lane + jnp.full((L,), h * L, jnp.int32)
        lat = plsc.load_gather(rowbuf, [rvec, zero_i])
        lon = plsc.load_gather(rowbuf, [rvec, one_i])
        plsc.store_scatter(gpsbuf, [rvec, zero_i], lat, mask=ones_b)
        plsc.store_scatter(gpsbuf, [rvec, one_i], lon, mask=ones_b)
    pltpu.sync_copy(gpsbuf, out_hbm.at[pl.ds(row0, ROWS)])


def kernel(x, gps_table):
    xt = lax.slice(x, (0, VA), (B, V))
    g128 = jnp.pad(gps_table, ((0, 0), (0, 126)))
    x_out, gps = _build_head()(x, xt, g128)
    return (x_out, gps)


# speedup vs baseline: 1.0003x; 1.0003x over previous
import functools

import jax
import jax.numpy as jnp
from jax import lax
from jax.experimental import pallas as pl
from jax.experimental.pallas import tpu as pltpu
from jax.experimental.pallas import tpu_sc as plsc

B = 1024
V = 100000
NC, NS, L = 2, 16, 16
NW = NC * NS
ROWS = B // NW
RG = 8
NG = ROWS // RG
VA = (V // 128) * 128
VT = V - VA
CW = 11 * 128
NCH = VA // CW
NU = NG * NCH
KS = CW // 32
NB = 4


@functools.cache
def _build_head():
    mesh = plsc.VectorSubcoreMesh(core_axis_name="c", subcore_axis_name="s",
                                  num_cores=NC, num_subcores=NS)
    return functools.partial(
        pl.kernel,
        out_type=(jax.ShapeDtypeStruct((B, V), jnp.float32),
                  jax.ShapeDtypeStruct((B, 2), jnp.float32)),
        mesh=mesh,
        compiler_params=pltpu.CompilerParams(needs_layout_passes=False,
                                             use_tc_tiling_on_sc=True),
        scratch_types=[
            [pltpu.VMEM((RG, CW), jnp.float32) for _ in range(NB)],
            pltpu.VMEM((ROWS, VT), jnp.float32),
            pltpu.VMEM((ROWS,), jnp.int32),
            pltpu.VMEM((ROWS, 2), jnp.float32),
            pltpu.VMEM((ROWS, 128), jnp.float32),
            [pltpu.SemaphoreType.DMA for _ in range(NB)],
            [pltpu.SemaphoreType.DMA for _ in range(NB)],
        ],
    )(_head_body)


def _head_body(x_hbm, xt_hbm, gps_hbm, xout_hbm, out_hbm, bufs, xtbuf,
               idxbuf, gpsbuf, rowbuf, isems, wsems):
    wid = lax.axis_index("s") * NC + lax.axis_index("c")
    row0 = wid * ROWS
    lane = lax.iota(jnp.int32, L)

    def dma_in(u, b):
        g, c = u // NCH, u % NCH
        return pltpu.make_async_copy(
            x_hbm.at[pl.ds(row0 + g * RG, RG), pl.ds(c * CW, CW)],
            bufs[b], isems[b])

    def dma_out(u, b):
        g, c = u // NCH, u % NCH
        return pltpu.make_async_copy(
            bufs[b],
            xout_hbm.at[pl.ds(row0 + g * RG, RG), pl.ds(c * CW, CW)],
            wsems[b])

    neg = jnp.full((L,), -jnp.inf, jnp.float32)
    zero = jnp.zeros((L,), jnp.int32)
    init_acc = ((neg, neg), (zero, zero))

    def scan_chunk(buf, c, acc):
        def body(k, a):
            iv = jnp.full((L,), c * KS + k, jnp.int32)
            out = []
            for r in range(RG):
                (v0, v1), (t0, t1) = a[r]
                x0 = buf[r, pl.ds(k * 32, L)]
                x1 = buf[r, pl.ds(k * 32 + L, L)]
                m0 = x0 > v0
                m1 = x1 > v1
                out.append(((jnp.where(m0, x0, v0), jnp.where(m1, x1, v1)),
                            (jnp.where(m0, iv, t0), jnp.where(m1, iv, t1))))
            return tuple(out)

        return lax.fori_loop(0, KS, body, acc)

    def epilogue(g, acc):
        for r in range(RG):
            row_l = g * RG + r
            (v0, v1), (t0, t1) = acc[r]
            pairs = [
                (v0, t0 * 32 + lane),
                (v1, t1 * 32 + (lane + L)),
                (xtbuf[row_l, pl.ds(0, L)], lane + VA),
                (xtbuf[row_l, pl.ds(L, L)], lane + (VA + L)),
            ]
            bv, bi = pairs[0]
            for v, i in pairs[1:]:
                take = (v > bv) | ((v == bv) & (i < bi))
                bv = jnp.where(take, v, bv)
                bi = jnp.where(take, i, bi)
            m = jnp.float32(-jnp.inf)
            mi = jnp.int32(V)
            for l in range(L):
                v, i = bv[l], bi[l]
                take = (v > m) | ((v == m) & (i < mi))
                m = jnp.where(take, v, m)
                mi = jnp.where(take, i, mi)
            plsc.store_scatter(idxbuf, [jnp.full((L,), row_l, jnp.int32)],
                               jnp.full((L,), mi, jnp.int32), mask=lane == 0)

    pltpu.sync_copy(xt_hbm.at[pl.ds(row0, ROWS)], xtbuf)
    dma_in(0, 0).start()
    dma_in(1, 1).start()

    def unit(u, j, acc):
        g, c = u // NCH, u % NCH
        dma_in(u, j).wait()
        acc = scan_chunk(bufs[j], c, acc)
        dma_out(u, j).start()

        jn = (j + 2) % NB

        @pl.when((u >= 2) & (u + 2 < NU))
        def _drain_write():
            dma_out(u - 2, jn).wait()

        @pl.when(u + 2 < NU)
        def _prefetch():
            dma_in(u + 2, jn).start()

        @pl.when(c == NCH - 1)
        def _finish():
            epilogue(g, acc)

        reset = jnp.full((L,), c == NCH - 1)
        return tuple(
            ((jnp.where(reset, neg, v0), jnp.where(reset, neg, v1)),
             (jnp.where(reset, zero, t0), jnp.where(reset, zero, t1)))
            for (v0, v1), (t0, t1) in acc)

    def quad_body(i, acc):
        for j in range(NB):
            acc = unit(NB * i + j, j, acc)
        return acc

    lax.fori_loop(0, NU // NB, quad_body, (init_acc,) * RG)

    for u in range(NU - NB, NU):
        dma_out(u, u % NB).wait()
    pltpu.sync_copy(xtbuf, xout_hbm.at[pl.ds(row0, ROWS), pl.ds(VA, VT)])

    pltpu.sync_copy(gps_hbm.at[idxbuf], rowbuf)
    ones_b = jnp.full((L,), True)
    zero_i = jnp.zeros((L,), jnp.int32)
    one_i = jnp.full((L,), 1, jnp.int32)
    for h in range(2):
        rvec = lane + jnp.full((L,), h * L, jnp.int32)
        lat = plsc.load_gather(rowbuf, [rvec, zero_i])
        lon = plsc.load_gather(rowbuf, [rvec, one_i])
        plsc.store_scatter(gpsbuf, [rvec, zero_i], lat, mask=ones_b)
        plsc.store_scatter(gpsbuf, [rvec, one_i], lon, mask=ones_b)
    pltpu.sync_copy(gpsbuf, out_hbm.at[pl.ds(row0, ROWS)])


def kernel(x, gps_table):
    xt = lax.slice(x, (0, VA), (B, V))
    g128 = jnp.pad(gps_table, ((0, 0), (0, 126)))
    x_out, gps = _build_head()(x, xt, g128)
    return (x_out, gps)


# speedup vs baseline: 3.0127x; 3.0118x over previous
import functools

import jax
import jax.numpy as jnp
from jax import lax
from jax.experimental import pallas as pl
from jax.experimental.pallas import tpu as pltpu
from jax.experimental.pallas import tpu_sc as plsc

B = 1024
V = 100000
NC, NS, L = 2, 16, 16
NQ = 4
VS = V // NQ
CW = 200
NU = VS // CW
GB = 128
NGRP = GB // L


@functools.cache
def _build_head():
    mesh = plsc.VectorSubcoreMesh(core_axis_name="c", subcore_axis_name="s",
                                  num_cores=NC, num_subcores=NS)
    return functools.partial(
        pl.kernel,
        out_type=(jax.ShapeDtypeStruct((V, B), jnp.float32),
                  jax.ShapeDtypeStruct((B, 2), jnp.float32)),
        mesh=mesh,
        compiler_params=pltpu.CompilerParams(needs_layout_passes=False,
                                             use_tc_tiling_on_sc=True),
        scratch_types=[
            pltpu.VMEM((CW, GB), jnp.float32),
            pltpu.VMEM((CW, GB), jnp.float32),
            pltpu.VMEM((GB,), jnp.float32),
            pltpu.VMEM((GB,), jnp.int32),
            pltpu.VMEM((NQ, GB), jnp.float32),
            pltpu.VMEM((NQ, GB), jnp.int32),
            pltpu.VMEM((GB,), jnp.int32),
            pltpu.VMEM((GB, 128), jnp.float32),
            pltpu.VMEM((GB, 2), jnp.float32),
            pltpu.VMEM_SHARED((NS, GB), jnp.float32),
            pltpu.VMEM_SHARED((NS, GB), jnp.int32),
            pltpu.SemaphoreType.DMA,
            pltpu.SemaphoreType.DMA,
            pltpu.SemaphoreType.DMA,
            pltpu.SemaphoreType.DMA,
        ],
    )(_head_body)


def _head_body(xT_hbm, gps_hbm, xout_hbm, out_hbm, buf0, buf1, myv, myi,
               mv, mi, gidx, growbuf, gout, shv, shi,
               isem0, isem1, wsem0, wsem1):
    cid = lax.axis_index("c")
    sid = lax.axis_index("s")
    p = sid % NQ
    q = cid * NQ + sid // NQ
    v0 = p * VS
    b0 = q * GB
    lane = lax.iota(jnp.int32, L)

    bufs = (buf0, buf1)
    isems = (isem0, isem1)
    wsems = (wsem0, wsem1)

    def dma_in(u, j):
        return pltpu.make_async_copy(
            xT_hbm.at[pl.ds(v0 + u * CW, CW), pl.ds(b0, GB)],
            bufs[j], isems[j])

    def dma_out(u, j):
        return pltpu.make_async_copy(
            bufs[j],
            xout_hbm.at[pl.ds(v0 + u * CW, CW), pl.ds(b0, GB)],
            wsems[j])

    neg = jnp.full((L,), -jnp.inf, jnp.float32)
    zero = jnp.zeros((L,), jnp.int32)

    def scan_chunk(buf, u, acc):
        def body(k2, a):
            out = list(a)
            for kk in range(2):
                k = k2 * 2 + kk
                iv = jnp.full((L,), u * CW + k, jnp.int32)
                nxt = []
                for g in range(NGRP):
                    v, t = out[g]
                    xv = buf[k, pl.ds(g * L, L)]
                    m = xv > v
                    nxt.append((jnp.where(m, xv, v), jnp.where(m, iv, t)))
                out = nxt
            return tuple(out)

        return lax.fori_loop(0, CW // 2, body, acc)

    def unit(u, j, acc):
        dma_in(u, j).wait()
        dma_out(u, j).start()
        acc = scan_chunk(bufs[j], u, acc)

        @pl.when(u + 2 < NU)
        def _roll():
            dma_out(u, j).wait()
            dma_in(u + 2, j).start()

        return acc

    dma_in(0, 0).start()
    dma_in(1, 1).start()

    def pair_body(i, acc):
        acc = unit(2 * i, 0, acc)
        acc = unit(2 * i + 1, 1, acc)
        return acc

    acc = lax.fori_loop(0, (NU - 1) // 2, pair_body,
                        ((neg, zero),) * NGRP)
    acc = unit(NU - 1, (NU - 1) % 2, acc)
    dma_out(NU - 2, (NU - 2) % 2).wait()
    dma_out(NU - 1, (NU - 1) % 2).wait()

    for g in range(NGRP):
        v, t = acc[g]
        myv[pl.ds(g * L, L)] = v
        myi[pl.ds(g * L, L)] = t + jnp.full((L,), v0, jnp.int32)
    pltpu.sync_copy(myv, shv.at[sid])
    pltpu.sync_copy(myi, shi.at[sid])
    plsc.subcore_barrier()

    @pl.when(p == 0)
    def _finish():
        pltpu.sync_copy(shv.at[pl.ds(sid, NQ)], mv)
        pltpu.sync_copy(shi.at[pl.ds(sid, NQ)], mi)
        for g in range(NGRP):
            bv = mv[0, pl.ds(g * L, L)]
            bi = mi[0, pl.ds(g * L, L)]
            for w in range(1, NQ):
                v = mv[w, pl.ds(g * L, L)]
                i = mi[w, pl.ds(g * L, L)]
                take = (v > bv) | ((v == bv) & (i < bi))
                bv = jnp.where(take, v, bv)
                bi = jnp.where(take, i, bi)
            gidx[pl.ds(g * L, L)] = bi

        pltpu.sync_copy(gps_hbm.at[gidx], growbuf)
        ones_b = jnp.full((L,), True)
        zero_i = jnp.zeros((L,), jnp.int32)
        one_i = jnp.full((L,), 1, jnp.int32)
        for g in range(NGRP):
            rvec = lane + jnp.full((L,), g * L, jnp.int32)
            lat = plsc.load_gather(growbuf, [rvec, zero_i])
            lon = plsc.load_gather(growbuf, [rvec, one_i])
            plsc.store_scatter(gout, [rvec, zero_i], lat, mask=ones_b)
            plsc.store_scatter(gout, [rvec, one_i], lon, mask=ones_b)
        pltpu.sync_copy(gout, out_hbm.at[pl.ds(b0, GB)])


def kernel(x, gps_table):
    xT = jnp.transpose(x)
    g128 = jnp.pad(gps_table, ((0, 0), (0, 126)))
    xTo, gps = _build_head()(xT, g128)
    return (jnp.transpose(xTo), gps)
